# baseline (speedup 1.0000x reference)
import functools

import jax
import jax.numpy as jnp
from jax import lax
from jax.experimental import pallas as pl
from jax.experimental.pallas import tpu as pltpu
from jax.experimental.pallas import tpu_sc as plsc

_NC = 2
_NS = 16
_NW = _NC * _NS
_L = 16


def _sc_count_body(p_hbm, u_hbm, out_hbm, pbuf, buf0, buf1, obuf, sem0, sem1,
                   *, rows_per_worker, chunk_rows):
    c = lax.axis_index("c")
    s = lax.axis_index("s")
    wid = s * _NC + c
    base = pl.multiple_of(wid * rows_per_worker, 8)
    nchunk = rows_per_worker // chunk_rows
    npair = nchunk // 2

    pltpu.sync_copy(p_hbm, pbuf)
    p = pbuf[...]

    def copy(k, buf, sem):
        return pltpu.make_async_copy(
            u_hbm.at[pl.ds(base + k * chunk_rows, chunk_rows)], buf, sem)

    def process(buf, a):
        def row_body(i, aa):
            u = buf[i]
            return aa + jnp.where(u < p, 1.0, 0.0)
        return lax.fori_loop(0, chunk_rows, row_body, a, unroll=8)

    copy(0, buf0, sem0).start()
    copy(1, buf1, sem1).start()

    def pair_body(t, a):
        k0 = 2 * t
        copy(k0, buf0, sem0).wait()
        a = process(buf0, a)

        @pl.when(t < npair - 1)
        def _():
            copy(k0 + 2, buf0, sem0).start()

        copy(k0 + 1, buf1, sem1).wait()
        a = process(buf1, a)

        @pl.when(t < npair - 1)
        def _():
            copy(k0 + 3, buf1, sem1).start()

        return a

    acc = lax.fori_loop(0, npair, pair_body, jnp.zeros((_L,), jnp.float32))

    obuf[...] = acc
    pltpu.sync_copy(obuf, out_hbm.at[wid])


def _finish_body(m_ref, p_ref, tail_ref, ops_ref, o_ref, *, s_total):
    m = jnp.sum(m_ref[...], axis=0, keepdims=True)
    tail = (tail_ref[...] < p_ref[...]).astype(jnp.float32)
    m = m + jnp.sum(tail, axis=0, keepdims=True)
    v = s_total - 2.0 * m
    o_ref[...] = jnp.sum(ops_ref[...] * v, axis=1, keepdims=True) * (1.0 / s_total)


def kernel(logits, uniforms, ops):
    s_total, n_bits = uniforms.shape
    num_ops = ops.shape[0]
    rpw = (s_total // (8 * _NW)) * 8
    main_rows = rpw * _NW
    tail_rows = s_total - main_rows
    chunk_rows = 504
    assert rpw % chunk_rows == 0 and chunk_rows % 8 == 0
    assert (rpw // chunk_rows) % 2 == 0

    probs = jax.nn.sigmoid(logits)
    u_tail = lax.slice(uniforms, (main_rows, 0), (s_total, n_bits))

    mesh = plsc.VectorSubcoreMesh(
        core_axis_name="c", subcore_axis_name="s",
        num_cores=_NC, num_subcores=_NS)
    sc_fn = pl.kernel(
        functools.partial(_sc_count_body,
                          rows_per_worker=rpw, chunk_rows=chunk_rows),
        out_type=jax.ShapeDtypeStruct((_NW, _L), jnp.float32),
        mesh=mesh,
        scratch_types=[
            pltpu.VMEM((_L,), jnp.float32),
            pltpu.VMEM((chunk_rows, _L), jnp.float32),
            pltpu.VMEM((chunk_rows, _L), jnp.float32),
            pltpu.VMEM((_L,), jnp.float32),
            pltpu.SemaphoreType.DMA,
            pltpu.SemaphoreType.DMA,
        ],
        compiler_params=pltpu.CompilerParams(
            use_tc_tiling_on_sc=False, needs_layout_passes=False),
    )
    m32 = sc_fn(probs, uniforms)

    out = pl.pallas_call(
        functools.partial(_finish_body, s_total=float(s_total)),
        out_shape=jax.ShapeDtypeStruct((num_ops, 1), jnp.float32),
    )(m32, probs.reshape(1, n_bits), u_tail, ops)
    return out.reshape(num_ops)


# speedup vs baseline: 7.5508x; 7.5508x over previous
import functools

import jax
import jax.numpy as jnp
from jax.experimental import pallas as pl
from jax.experimental.pallas import tpu as pltpu


def _count_body(p_ref, u_ref, opst_ref, o_ref, acc_ref, *, s_total):
    i = pl.program_id(0)
    n = pl.num_programs(0)

    @pl.when(i == 0)
    def _init():
        acc_ref[...] = jnp.zeros_like(acc_ref)

    u = u_ref[...]
    lt = (u < p_ref[...]).astype(jnp.float32)
    acc_ref[...] += jnp.sum(lt, axis=1, keepdims=True)

    @pl.when(i == n - 1)
    def _finish():
        v = s_total - 2.0 * acc_ref[...]
        o_ref[...] = jnp.sum(opst_ref[...] * v, axis=0, keepdims=True) * (
            1.0 / s_total)


def kernel(logits, uniforms, ops):
    s_total, n_bits = uniforms.shape
    num_ops = ops.shape[0]

    probs = jax.nn.sigmoid(logits)
    padded = 1_024_000
    u_t = jnp.pad(uniforms.T, ((0, 0), (0, padded - s_total)),
                  constant_values=2.0)

    c_block = 64_000
    grid = (padded // c_block,)
    out = pl.pallas_call(
        functools.partial(_count_body, s_total=float(s_total)),
        grid=grid,
        in_specs=[
            pl.BlockSpec((n_bits, 1), lambda i: (0, 0)),
            pl.BlockSpec((n_bits, c_block), lambda i: (0, i)),
            pl.BlockSpec((n_bits, num_ops), lambda i: (0, 0)),
        ],
        out_specs=pl.BlockSpec((1, num_ops), lambda i: (0, 0)),
        out_shape=jax.ShapeDtypeStruct((1, num_ops), jnp.float32),
        scratch_shapes=[pltpu.VMEM((n_bits, 1), jnp.float32)],
    )(probs.reshape(n_bits, 1), u_t, ops.T)
    return out.reshape(num_ops)


# speedup vs baseline: 7.7345x; 1.0243x over previous
import functools

import jax
import jax.numpy as jnp
from jax.experimental import pallas as pl
from jax.experimental.pallas import tpu as pltpu


def _count_body(p_ref, u_ref, opst_ref, o_ref, acc_ref, *, s_total):
    i = pl.program_id(0)
    n = pl.num_programs(0)

    @pl.when(i == 0)
    def _init():
        acc_ref[...] = jnp.zeros_like(acc_ref)

    u = u_ref[...]
    lt = (u < p_ref[...]).astype(jnp.float32)
    acc_ref[...] += jnp.sum(lt, axis=1, keepdims=True)

    @pl.when(i == n - 1)
    def _finish():
        v = s_total - 2.0 * acc_ref[...]
        o_ref[...] = jnp.sum(opst_ref[...] * v, axis=0, keepdims=True) * (
            1.0 / s_total)


def kernel(logits, uniforms, ops):
    s_total, n_bits = uniforms.shape
    num_ops = ops.shape[0]

    probs = jax.nn.sigmoid(logits)
    padded = 1_000_064
    u_t = jnp.pad(uniforms.T, ((0, 0), (0, padded - s_total)),
                  constant_values=2.0)

    c_block = 76_928
    grid = (padded // c_block,)
    out = pl.pallas_call(
        functools.partial(_count_body, s_total=float(s_total)),
        grid=grid,
        in_specs=[
            pl.BlockSpec((n_bits, 1), lambda i: (0, 0)),
            pl.BlockSpec((n_bits, c_block), lambda i: (0, i)),
            pl.BlockSpec((n_bits, num_ops), lambda i: (0, 0)),
        ],
        out_specs=pl.BlockSpec((1, num_ops), lambda i: (0, 0)),
        out_shape=jax.ShapeDtypeStruct((1, num_ops), jnp.float32),
        scratch_shapes=[pltpu.VMEM((n_bits, 1), jnp.float32)],
    )(probs.reshape(n_bits, 1), u_t, ops.T)
    return out.reshape(num_ops)
